# baseline (speedup 1.0000x reference)
import functools

import jax
import jax.numpy as jnp
from jax.experimental import pallas as pl
from jax.experimental.pallas import tpu as pltpu
from jax.experimental.pallas import tpu_sc as plsc

_BETA = 0.25


def _dist_body(nb, kb, bn, bk, n_tokens, x_ref, e_ref, arg_ref, min_ref,
               loss_ref):
    n = pl.program_id(0)
    kk = pl.program_id(1)
    x = x_ref[...]
    e = e_ref[pl.ds(kk * bk, bk), :]
    s = jax.lax.dot_general(x, e, (((1,), (1,)), ((), ())),
                            preferred_element_type=jnp.float32)
    x_sq = jnp.sum(x * x, axis=1, keepdims=True)
    e_sq = jnp.sum(e * e, axis=1)[None, :]
    dist = (x_sq + e_sq) - 2.0 * s

    m = jnp.min(dist, axis=1, keepdims=True)
    lanes = jax.lax.broadcasted_iota(jnp.int32, dist.shape, 1)
    masked = jnp.where(dist == m, lanes, jnp.int32(bk))
    a = jnp.min(masked, axis=1, keepdims=True) + kk * bk

    @pl.when(kk == 0)
    def _():
        min_ref[...] = m
        arg_ref[...] = a

    @pl.when(kk > 0)
    def _():
        prev = min_ref[...]
        upd = m < prev
        min_ref[...] = jnp.where(upd, m, prev)
        arg_ref[...] = jnp.where(upd, a, arg_ref[...])

    @pl.when(kk == kb - 1)
    def _():
        part = jnp.sum(min_ref[...], keepdims=True).reshape(1, 1)
        prev = jnp.where(n == 0, jnp.zeros((1, 1), jnp.float32), loss_ref[...])
        tot = prev + part
        loss_ref[...] = jnp.where(n == nb - 1, tot * (_BETA / n_tokens), tot)


def _argmin_min_loss(X, E_weight, bn=256, bk=1024, interpret=False):
    n_tokens, d = X.shape
    k_codes = E_weight.shape[0]
    nb, kb = n_tokens // bn, k_codes // bk
    body = functools.partial(_dist_body, nb, kb, bn, bk, n_tokens)
    return pl.pallas_call(
        body,
        grid=(nb, kb),
        in_specs=[
            pl.BlockSpec((bn, d), lambda n, k: (n, 0)),
            pl.BlockSpec((k_codes, d), lambda n, k: (0, 0)),
        ],
        out_specs=[
            pl.BlockSpec((bn, 1), lambda n, k: (n, 0)),
            pl.BlockSpec((bn, 1), lambda n, k: (n, 0)),
            pl.BlockSpec((1, 1), lambda n, k: (0, 0)),
        ],
        out_shape=[
            jax.ShapeDtypeStruct((n_tokens, 1), jnp.int32),
            jax.ShapeDtypeStruct((n_tokens, 1), jnp.float32),
            jax.ShapeDtypeStruct((1, 1), jnp.float32),
        ],
        compiler_params=pltpu.CompilerParams(
            dimension_semantics=("arbitrary", "arbitrary")),
        interpret=interpret,
    )(X, E_weight)


def _gather_rows(E_weight, argmins, window=128):
    n_tokens = argmins.shape[0]
    d = E_weight.shape[1]
    idx2 = argmins.reshape(1, n_tokens)
    mesh = plsc.VectorSubcoreMesh(core_axis_name="c", subcore_axis_name="s")

    @pl.kernel(out_type=jax.ShapeDtypeStruct((n_tokens, d), E_weight.dtype),
               mesh=mesh)
    def gather_kernel(e_hbm, i_hbm, o_hbm):
        def body(i_vmem, o_vmem):
            pltpu.sync_copy(e_hbm.at[i_vmem.at[0]], o_vmem)

        pltpu.emit_pipeline(
            body,
            grid=(n_tokens // window,),
            in_specs=[pl.BlockSpec((1, window), index_map=lambda i: (0, i))],
            out_specs=[pl.BlockSpec((window, d), index_map=lambda i: (i, 0))],
            core_axis_name=("c", "s"),
            dimension_semantics=(pltpu.PARALLEL,),
        )(i_hbm, o_hbm)

    return gather_kernel(E_weight, idx2)


def kernel(X, E_weight):
    n_tokens = X.shape[0]
    arg2, min2, loss2 = _argmin_min_loss(X, E_weight)
    argmins = arg2.reshape(n_tokens)
    min_dist = min2.reshape(n_tokens)
    loss = loss2[0, 0]
    z_st = _gather_rows(E_weight, argmins)
    return (z_st, loss, argmins, min_dist)
